# baseline (speedup 1.0000x reference)
import functools

import jax
import jax.numpy as jnp
from jax.experimental import pallas as pl
from jax.experimental.pallas import tpu as pltpu


def _sigmoid(x):
    return 0.5 * (jnp.tanh(0.5 * x) + 1.0)


def _round_up(v, m):
    return ((v + m - 1) // m) * m


def _make_body(TB, S, L, pad, K, C_in, C1, C2, H, F1):
    Lp = L + 2 * pad
    TBs = TB // S

    def body(xf_ref, w1e_ref, b1_ref, w2e_ref, b2_ref,
             wih_ref, whh_ref, bl_ref, wfc1_ref, bfc1_ref,
             wfc2_ref, bfc2_ref, alpha_ref, out_ref):
        f32 = jnp.float32
        bf16 = jnp.bfloat16
        x2 = xf_ref[...]
        w1e = w1e_ref[...]
        w2e = w2e_ref[...]

        def lanes(j):
            return slice(j * TB, (j + 1) * TB)

        p1 = jnp.dot(w1e, x2, preferred_element_type=f32)
        b1c = b1_ref[...]
        zc1 = jnp.zeros((C1, pad * TB), bf16)
        pieces = [zc1]
        for l in range(L):
            acc = p1[0:C1, lanes(l)]
            for k in range(1, K):
                acc = acc + p1[k * C1:(k + 1) * C1, lanes(l + k)]
            pieces.append(jnp.maximum(acc + b1c, 0.0).astype(bf16))
        pieces.append(zc1)
        h1p = jnp.concatenate(pieces, axis=1)
        p2 = jnp.dot(w2e, h1p, preferred_element_type=f32)
        b2c = b2_ref[...]

        whh = whh_ref[...]
        blc = bl_ref[...]
        xg = jnp.dot(wih_ref[...], x2, preferred_element_type=f32) + blc
        hs = [jnp.zeros((H, TBs), f32) for _ in range(S)]
        cs = [jnp.zeros((H, TBs), f32) for _ in range(S)]
        pool = None
        for t in range(L):
            base = (t + pad) * TB
            for s in range(S):
                g = (xg[:, base + s * TBs:base + (s + 1) * TBs] +
                     jnp.dot(whh, hs[s].astype(bf16), preferred_element_type=f32))
                s_if = _sigmoid(g[0:2 * H])
                s_o = _sigmoid(g[3 * H:4 * H])
                t_g = jnp.tanh(g[2 * H:3 * H])
                cs[s] = s_if[H:2 * H] * cs[s] + s_if[0:H] * t_g
                hs[s] = s_o * jnp.tanh(cs[s])
            acc = p2[0:C2, lanes(t)]
            for k in range(1, K):
                acc = acc + p2[k * C2:(k + 1) * C2, lanes(t + k)]
            h2 = jnp.maximum(acc + b2c, 0.0)
            pool = h2 if pool is None else pool + h2
        rnn_t = jnp.concatenate(hs, axis=1) if S > 1 else hs[0]
        cnn_t = pool * (1.0 / L)

        a = alpha_ref[0]
        fused = a * cnn_t + (1.0 - a) * rnn_t
        z1 = jnp.dot(wfc1_ref[...], fused.astype(bf16),
                     preferred_element_type=f32) + bfc1_ref[...]
        z1 = jnp.maximum(z1, 0.0)
        z2 = jnp.dot(wfc2_ref[...], z1.astype(bf16),
                     preferred_element_type=f32) + bfc2_ref[...]
        out_ref[...] = _sigmoid(z2).reshape(1, 1, TB)

    return body


@functools.partial(jax.jit, static_argnames=())
def kernel(x, w1, b1, w2, b2, w_ih, w_hh, b_ih, b_hh,
           alpha, wfc1, bfc1, wfc2, bfc2):
    B, C_in, L = x.shape
    C1, _, K = w1.shape
    C2 = w2.shape[0]
    H = w_hh.shape[1]
    F1 = wfc1.shape[0]
    pad = K // 2
    Lp = L + 2 * pad

    TB = 512 if B >= 1024 else max(128, _round_up(B, 128))
    S = TB // 128
    B_pad = _round_up(B, TB)
    num_tiles = B_pad // TB

    bf16 = jnp.bfloat16
    xf = jnp.pad(x.astype(bf16), ((0, B_pad - B), (0, 0), (pad, pad)))
    xf = xf.reshape(num_tiles, TB, C_in, Lp)
    xf = jnp.transpose(xf, (2, 0, 3, 1)).reshape(C_in, num_tiles * Lp * TB)

    w1e = jnp.transpose(w1, (2, 0, 1)).reshape(K * C1, C_in).astype(bf16)
    w2e = jnp.transpose(w2, (2, 0, 1)).reshape(K * C2, C1).astype(bf16)
    b1c = b1.reshape(C1, 1)
    b2c = b2.reshape(C2, 1)
    wih = w_ih.astype(bf16)
    whh = w_hh.astype(bf16)
    blc = (b_ih + b_hh).reshape(4 * H, 1)
    wfc1_b = wfc1.astype(bf16)
    bfc1c = bfc1.reshape(F1, 1)
    wfc2_b = wfc2.astype(bf16)
    bfc2c = bfc2.reshape(1, 1)
    alpha_s = jnp.asarray(alpha, jnp.float32).reshape(1)

    body = _make_body(TB, S, L, pad, K, C_in, C1, C2, H, F1)

    out = pl.pallas_call(
        body,
        out_shape=jax.ShapeDtypeStruct((num_tiles, 1, TB), jnp.float32),
        grid_spec=pltpu.PrefetchScalarGridSpec(
            num_scalar_prefetch=0,
            grid=(num_tiles,),
            in_specs=[
                pl.BlockSpec((C_in, Lp * TB), lambda i: (0, i)),
                pl.BlockSpec((K * C1, C_in), lambda i: (0, 0)),
                pl.BlockSpec((C1, 1), lambda i: (0, 0)),
                pl.BlockSpec((K * C2, C1), lambda i: (0, 0)),
                pl.BlockSpec((C2, 1), lambda i: (0, 0)),
                pl.BlockSpec((4 * H, C_in), lambda i: (0, 0)),
                pl.BlockSpec((4 * H, H), lambda i: (0, 0)),
                pl.BlockSpec((4 * H, 1), lambda i: (0, 0)),
                pl.BlockSpec((F1, C2), lambda i: (0, 0)),
                pl.BlockSpec((F1, 1), lambda i: (0, 0)),
                pl.BlockSpec((1, F1), lambda i: (0, 0)),
                pl.BlockSpec((1, 1), lambda i: (0, 0)),
                pl.BlockSpec(memory_space=pltpu.MemorySpace.SMEM),
            ],
            out_specs=pl.BlockSpec((1, 1, TB), lambda i: (i, 0, 0)),
        ),
        compiler_params=pltpu.CompilerParams(
            dimension_semantics=("parallel",),
            vmem_limit_bytes=60 * 1024 * 1024,
        ),
    )(xf, w1e, b1c, w2e, b2c, wih, whh, blc,
      wfc1_b, bfc1c, wfc2_b, bfc2c, alpha_s)
    return out.reshape(B_pad, 1)[:B]


# speedup vs baseline: 1.3385x; 1.3385x over previous
import jax
import jax.numpy as jnp
from jax.experimental import pallas as pl
from jax.experimental.pallas import tpu as pltpu


def _sigmoid(x):
    return 0.5 * (jnp.tanh(0.5 * x) + 1.0)


def _round_up(v, m):
    return ((v + m - 1) // m) * m


def _make_conv_body(TB, L, pad, K, C1, C2):
    Lp = L + 2 * pad

    def body(x2_ref, w1e_ref, b1_ref, w2e_ref, b2_ref, out_ref):
        f32 = jnp.float32
        bf16 = jnp.bfloat16
        x2 = x2_ref[...]

        def lanes(j):
            return slice(j * TB, (j + 1) * TB)

        p1 = jnp.dot(w1e_ref[...], x2, preferred_element_type=f32)
        b1c = b1_ref[...]
        zc1 = jnp.zeros((C1, pad * TB), bf16)
        pieces = [zc1]
        for l in range(L):
            acc = p1[0:C1, lanes(l)]
            for k in range(1, K):
                acc = acc + p1[k * C1:(k + 1) * C1, lanes(l + k)]
            pieces.append(jnp.maximum(acc + b1c, 0.0).astype(bf16))
        pieces.append(zc1)
        h1p = jnp.concatenate(pieces, axis=1)
        p2 = jnp.dot(w2e_ref[...], h1p, preferred_element_type=f32)
        b2c = b2_ref[...]
        pool = None
        for l in range(L):
            acc = p2[0:C2, lanes(l)]
            for k in range(1, K):
                acc = acc + p2[k * C2:(k + 1) * C2, lanes(l + k)]
            h2 = jnp.maximum(acc + b2c, 0.0)
            pool = h2 if pool is None else pool + h2
        out_ref[...] = (pool * (1.0 / L)).reshape(1, C2, TB)

    return body


def _make_rnn_body(TB, R, S, L, pad, K, C_in, H, C2, F1):
    Lp = L + 2 * pad
    TBc = TB // R
    TBs = TB // S
    CPH = S // R

    def body(x2_ref, wih_ref, whh_ref, bl_ref, cnn_ref,
             wfc1_ref, bfc1_ref, wfc2_ref, bfc2_ref, alpha_ref, out_ref):
        f32 = jnp.float32
        bf16 = jnp.bfloat16
        x2 = x2_ref[...]
        whh = whh_ref[...]
        blc = bl_ref[...]
        xg = jnp.dot(wih_ref[...], x2, preferred_element_type=f32) + blc

        def chain_lane(s, t):
            half, c = divmod(s, CPH)
            return half * Lp * TBc + (t + pad) * TBc + c * TBs

        hs = [jnp.zeros((H, TBs), f32) for _ in range(S)]
        cs = [jnp.zeros((H, TBs), f32) for _ in range(S)]
        for t in range(L):
            for s in range(S):
                base = chain_lane(s, t)
                g = (xg[:, base:base + TBs] +
                     jnp.dot(whh, hs[s].astype(bf16), preferred_element_type=f32))
                s_if = _sigmoid(g[0:2 * H])
                s_o = _sigmoid(g[3 * H:4 * H])
                t_g = jnp.tanh(g[2 * H:3 * H])
                cs[s] = s_if[H:2 * H] * cs[s] + s_if[0:H] * t_g
                hs[s] = s_o * jnp.tanh(cs[s])
        rnn_t = jnp.concatenate(hs, axis=1)
        cnn_t = jnp.concatenate(
            [cnn_ref[r] for r in range(R)], axis=1)

        a = alpha_ref[0]
        fused = a * cnn_t + (1.0 - a) * rnn_t
        z1 = jnp.dot(wfc1_ref[...], fused.astype(bf16),
                     preferred_element_type=f32) + bfc1_ref[...]
        z1 = jnp.maximum(z1, 0.0)
        z2 = jnp.dot(wfc2_ref[...], z1.astype(bf16),
                     preferred_element_type=f32) + bfc2_ref[...]
        out_ref[...] = _sigmoid(z2).reshape(1, 1, TB)

    return body


def kernel(x, w1, b1, w2, b2, w_ih, w_hh, b_ih, b_hh,
           alpha, wfc1, bfc1, wfc2, bfc2):
    B, C_in, L = x.shape
    C1, _, K = w1.shape
    C2 = w2.shape[0]
    H = w_hh.shape[1]
    F1 = wfc1.shape[0]
    pad = K // 2
    Lp = L + 2 * pad

    TBc = 512 if B >= 1024 else max(128, _round_up(B, 128))
    R = 2 if B >= 2 * TBc else 1
    TBl = R * TBc
    S = TBl // 128
    B_pad = _round_up(B, TBl)
    nt_c = B_pad // TBc
    nt_l = B_pad // TBl

    bf16 = jnp.bfloat16
    xf = jnp.pad(x.astype(bf16), ((0, B_pad - B), (0, 0), (pad, pad)))
    xf = xf.reshape(nt_c, TBc, C_in, Lp)
    xf = jnp.transpose(xf, (2, 0, 3, 1)).reshape(C_in, nt_c * Lp * TBc)

    w1e = jnp.transpose(w1, (2, 0, 1)).reshape(K * C1, C_in).astype(bf16)
    w2e = jnp.transpose(w2, (2, 0, 1)).reshape(K * C2, C1).astype(bf16)
    b1c = b1.reshape(C1, 1)
    b2c = b2.reshape(C2, 1)
    wih = w_ih.astype(bf16)
    whh = w_hh.astype(bf16)
    blc = (b_ih + b_hh).reshape(4 * H, 1)
    wfc1_b = wfc1.astype(bf16)
    bfc1c = bfc1.reshape(F1, 1)
    wfc2_b = wfc2.astype(bf16)
    bfc2c = bfc2.reshape(1, 1)
    alpha_s = jnp.asarray(alpha, jnp.float32).reshape(1)

    conv_body = _make_conv_body(TBc, L, pad, K, C1, C2)
    cnn = pl.pallas_call(
        conv_body,
        out_shape=jax.ShapeDtypeStruct((nt_c, C2, TBc), jnp.float32),
        grid_spec=pltpu.PrefetchScalarGridSpec(
            num_scalar_prefetch=0,
            grid=(nt_c,),
            in_specs=[
                pl.BlockSpec((C_in, Lp * TBc), lambda i: (0, i)),
                pl.BlockSpec((K * C1, C_in), lambda i: (0, 0)),
                pl.BlockSpec((C1, 1), lambda i: (0, 0)),
                pl.BlockSpec((K * C2, C1), lambda i: (0, 0)),
                pl.BlockSpec((C2, 1), lambda i: (0, 0)),
            ],
            out_specs=pl.BlockSpec((1, C2, TBc), lambda i: (i, 0, 0)),
        ),
        compiler_params=pltpu.CompilerParams(
            dimension_semantics=("parallel",),
            vmem_limit_bytes=52 * 1024 * 1024,
        ),
    )(xf, w1e, b1c, w2e, b2c)

    rnn_body = _make_rnn_body(TBl, R, TBl // 128 if TBl >= 128 else 1,
                              L, pad, K, C_in, H, C2, F1)
    out = pl.pallas_call(
        rnn_body,
        out_shape=jax.ShapeDtypeStruct((nt_l, 1, TBl), jnp.float32),
        grid_spec=pltpu.PrefetchScalarGridSpec(
            num_scalar_prefetch=0,
            grid=(nt_l,),
            in_specs=[
                pl.BlockSpec((C_in, R * Lp * TBc), lambda i: (0, i)),
                pl.BlockSpec((4 * H, C_in), lambda i: (0, 0)),
                pl.BlockSpec((4 * H, H), lambda i: (0, 0)),
                pl.BlockSpec((4 * H, 1), lambda i: (0, 0)),
                pl.BlockSpec((R, C2, TBc), lambda i: (i, 0, 0)),
                pl.BlockSpec((F1, C2), lambda i: (0, 0)),
                pl.BlockSpec((F1, 1), lambda i: (0, 0)),
                pl.BlockSpec((1, F1), lambda i: (0, 0)),
                pl.BlockSpec((1, 1), lambda i: (0, 0)),
                pl.BlockSpec(memory_space=pltpu.MemorySpace.SMEM),
            ],
            out_specs=pl.BlockSpec((1, 1, TBl), lambda i: (i, 0, 0)),
        ),
        compiler_params=pltpu.CompilerParams(
            dimension_semantics=("parallel",),
            vmem_limit_bytes=45 * 1024 * 1024,
        ),
    )(xf, wih, whh, blc, cnn, wfc1_b, bfc1c, wfc2_b, bfc2c, alpha_s)
    return out.reshape(B_pad, 1)[:B]


# speedup vs baseline: 1.5948x; 1.1915x over previous
import jax
import jax.numpy as jnp
from jax.experimental import pallas as pl
from jax.experimental.pallas import tpu as pltpu


def _sigmoid(x):
    return 0.5 * (jnp.tanh(0.5 * x) + 1.0)


def _round_up(v, m):
    return ((v + m - 1) // m) * m


def _tree_sum(xs):
    while len(xs) > 1:
        nxt = [xs[i] + xs[i + 1] for i in range(0, len(xs) - 1, 2)]
        if len(xs) % 2:
            nxt.append(xs[-1])
        xs = nxt
    return xs[0]


def _make_conv_body(TB, L, pad, K, C1, C2):
    Lp = L + 2 * pad

    def body(x2_ref, w1t_ref, b1_ref, w2t_ref, b2_ref, out_ref):
        f32 = jnp.float32
        bf16 = jnp.bfloat16
        x2 = x2_ref[...]

        h1 = _tree_sum([
            jnp.dot(w1t_ref[k], x2[:, k * TB:(k + L) * TB],
                    preferred_element_type=f32)
            for k in range(K)])
        h1 = jnp.maximum(h1 + b1_ref[...], 0.0).astype(bf16)
        zc1 = jnp.zeros((C1, pad * TB), bf16)
        h1p = jnp.concatenate([zc1, h1, zc1], axis=1)

        h2 = _tree_sum([
            jnp.dot(w2t_ref[k], h1p[:, k * TB:(k + L) * TB],
                    preferred_element_type=f32)
            for k in range(K)])
        h2 = jnp.maximum(h2 + b2_ref[...], 0.0)

        pool = _tree_sum([h2[:, l * TB:(l + 1) * TB] for l in range(L)])
        out_ref[...] = (pool * (1.0 / L)).reshape(1, C2, TB)

    return body


def _make_rnn_body(TB, S, L, pad, C8, H, C2, F1):
    Lp = L + 2 * pad
    TBs = TB // S

    def body(x8_ref, wcat_ref, cnn_ref, wfc1_ref, bfc1_ref,
             wfc2_ref, bfc2_ref, alpha_ref, out_ref):
        f32 = jnp.float32
        bf16 = jnp.bfloat16
        x8 = x8_ref[...]
        wcat = wcat_ref[...]
        ones_row = jnp.ones((8, TBs), bf16)

        hs = [jnp.zeros((H, TBs), f32) for _ in range(S)]
        cs = [jnp.zeros((H, TBs), f32) for _ in range(S)]
        for t in range(L):
            base = (t + pad) * TB
            for s in range(S):
                lo = base + s * TBs
                rhs = jnp.concatenate(
                    [x8[:, lo:lo + TBs], hs[s].astype(bf16), ones_row],
                    axis=0)
                g = jnp.dot(wcat, rhs, preferred_element_type=f32)
                s_if = _sigmoid(g[0:2 * H])
                s_o = _sigmoid(g[3 * H:4 * H])
                t_g = jnp.tanh(g[2 * H:3 * H])
                cs[s] = s_if[H:2 * H] * cs[s] + s_if[0:H] * t_g
                hs[s] = s_o * jnp.tanh(cs[s])
        rnn_t = jnp.concatenate(hs, axis=1)

        a = alpha_ref[0]
        fused = a * cnn_ref[0] + (1.0 - a) * rnn_t
        z1 = jnp.dot(wfc1_ref[...], fused.astype(bf16),
                     preferred_element_type=f32) + bfc1_ref[...]
        z1 = jnp.maximum(z1, 0.0)
        z2 = jnp.dot(wfc2_ref[...], z1.astype(bf16),
                     preferred_element_type=f32) + bfc2_ref[...]
        out_ref[...] = _sigmoid(z2).reshape(1, 1, TB)

    return body


def kernel(x, w1, b1, w2, b2, w_ih, w_hh, b_ih, b_hh,
           alpha, wfc1, bfc1, wfc2, bfc2):
    B, C_in, L = x.shape
    C1, _, K = w1.shape
    C2 = w2.shape[0]
    H = w_hh.shape[1]
    F1 = wfc1.shape[0]
    pad = K // 2
    Lp = L + 2 * pad

    TB = 1024 if B >= 2048 else max(128, _round_up(B, 128))
    S = max(1, TB // 128)
    B_pad = _round_up(B, TB)
    nt = B_pad // TB

    bf16 = jnp.bfloat16
    f32 = jnp.float32
    xp = jnp.pad(x.astype(bf16), ((0, B_pad - B), (0, 0), (pad, pad)))
    xp = xp.reshape(nt, TB, C_in, Lp)
    x2 = jnp.transpose(xp, (2, 0, 3, 1)).reshape(C_in, nt * Lp * TB)
    x8 = jnp.pad(xp, ((0, 0), (0, 0), (0, 8 - C_in), (0, 0)))
    x8 = jnp.transpose(x8, (2, 0, 3, 1)).reshape(8, nt * Lp * TB)

    w1t = jnp.transpose(w1, (2, 0, 1)).astype(bf16)
    w2t = jnp.transpose(w2, (2, 0, 1)).astype(bf16)
    b1c = b1.reshape(C1, 1)
    b2c = b2.reshape(C2, 1)
    wcat = jnp.concatenate([
        w_ih, jnp.zeros((4 * H, 8 - C_in), f32), w_hh,
        (b_ih + b_hh).reshape(4 * H, 1), jnp.zeros((4 * H, 7), f32),
    ], axis=1).astype(bf16)
    wfc1_b = wfc1.astype(bf16)
    bfc1c = bfc1.reshape(F1, 1)
    wfc2_b = wfc2.astype(bf16)
    bfc2c = bfc2.reshape(1, 1)
    alpha_s = jnp.asarray(alpha, f32).reshape(1)

    conv_body = _make_conv_body(TB, L, pad, K, C1, C2)
    cnn = pl.pallas_call(
        conv_body,
        out_shape=jax.ShapeDtypeStruct((nt, C2, TB), f32),
        grid_spec=pltpu.PrefetchScalarGridSpec(
            num_scalar_prefetch=0,
            grid=(nt,),
            in_specs=[
                pl.BlockSpec((C_in, Lp * TB), lambda i: (0, i)),
                pl.BlockSpec((K, C1, C_in), lambda i: (0, 0, 0)),
                pl.BlockSpec((C1, 1), lambda i: (0, 0)),
                pl.BlockSpec((K, C2, C1), lambda i: (0, 0, 0)),
                pl.BlockSpec((C2, 1), lambda i: (0, 0)),
            ],
            out_specs=pl.BlockSpec((1, C2, TB), lambda i: (i, 0, 0)),
        ),
        compiler_params=pltpu.CompilerParams(
            dimension_semantics=("parallel",),
            vmem_limit_bytes=48 * 1024 * 1024,
        ),
    )(x2, w1t, b1c, w2t, b2c)

    rnn_body = _make_rnn_body(TB, S, L, pad, 8, H, C2, F1)
    out = pl.pallas_call(
        rnn_body,
        out_shape=jax.ShapeDtypeStruct((nt, 1, TB), f32),
        grid_spec=pltpu.PrefetchScalarGridSpec(
            num_scalar_prefetch=0,
            grid=(nt,),
            in_specs=[
                pl.BlockSpec((8, Lp * TB), lambda i: (0, i)),
                pl.BlockSpec((4 * H, 48), lambda i: (0, 0)),
                pl.BlockSpec((1, C2, TB), lambda i: (i, 0, 0)),
                pl.BlockSpec((F1, C2), lambda i: (0, 0)),
                pl.BlockSpec((F1, 1), lambda i: (0, 0)),
                pl.BlockSpec((1, F1), lambda i: (0, 0)),
                pl.BlockSpec((1, 1), lambda i: (0, 0)),
                pl.BlockSpec(memory_space=pltpu.MemorySpace.SMEM),
            ],
            out_specs=pl.BlockSpec((1, 1, TB), lambda i: (i, 0, 0)),
        ),
        compiler_params=pltpu.CompilerParams(
            dimension_semantics=("parallel",),
            vmem_limit_bytes=45 * 1024 * 1024,
        ),
    )(x8, wcat, cnn, wfc1_b, bfc1c, wfc2_b, bfc2c, alpha_s)
    return out.reshape(B_pad, 1)[:B]


# speedup vs baseline: 2.0815x; 1.3052x over previous
import jax
import jax.numpy as jnp
from jax.experimental import pallas as pl
from jax.experimental.pallas import tpu as pltpu


def _sigmoid(x):
    return 0.5 * (jnp.tanh(0.5 * x) + 1.0)


def _round_up(v, m):
    return ((v + m - 1) // m) * m


def _tree_sum(xs):
    while len(xs) > 1:
        nxt = [xs[i] + xs[i + 1] for i in range(0, len(xs) - 1, 2)]
        if len(xs) % 2:
            nxt.append(xs[-1])
        xs = nxt
    return xs[0]


def _make_conv_body(TB, L, pad, K, C1, C2):
    Lp = L + 2 * pad

    def body(x2_ref, w1t_ref, b1_ref, w2t_ref, b2_ref, out_ref):
        f32 = jnp.float32
        bf16 = jnp.bfloat16
        x2 = x2_ref[...]

        h1 = _tree_sum([
            jnp.dot(w1t_ref[k], x2[:, k * TB:(k + L) * TB],
                    preferred_element_type=f32)
            for k in range(K)])
        h1 = jnp.maximum(h1 + b1_ref[...], 0.0).astype(bf16)
        zc1 = jnp.zeros((C1, pad * TB), bf16)
        h1p = jnp.concatenate([zc1, h1, zc1], axis=1)

        h2 = _tree_sum([
            jnp.dot(w2t_ref[k], h1p[:, k * TB:(k + L) * TB],
                    preferred_element_type=f32)
            for k in range(K)])
        h2 = jnp.maximum(h2 + b2_ref[...], 0.0)

        pool = _tree_sum([h2[:, l * TB:(l + 1) * TB] for l in range(L)])
        out_ref[...] = (pool * (1.0 / L)).reshape(1, C2, TB)

    return body


def _make_rnn_body(TB, S, L, pad, C8, H, C2, F1):
    Lp = L + 2 * pad
    TBs = TB // S

    def body(x8_ref, wcat_ref, cnn_ref, wfc1_ref, bfc1_ref,
             wfc2_ref, bfc2_ref, alpha_ref, out_ref):
        f32 = jnp.float32
        bf16 = jnp.bfloat16
        x8 = x8_ref[...]
        wcat = wcat_ref[...]
        ones_row = jnp.ones((8, TBs), bf16)

        hs = [jnp.zeros((H, TBs), f32) for _ in range(S)]
        cs = [jnp.zeros((H, TBs), f32) for _ in range(S)]
        for t in range(L):
            base = (t + pad) * TB
            for s in range(S):
                lo = base + s * TBs
                rhs = jnp.concatenate(
                    [x8[:, lo:lo + TBs], hs[s].astype(bf16), ones_row],
                    axis=0)
                g = jnp.dot(wcat, rhs, preferred_element_type=f32)
                s_if = _sigmoid(g[0:2 * H])
                s_o = _sigmoid(g[3 * H:4 * H])
                t_g = jnp.tanh(g[2 * H:3 * H])
                cs[s] = s_if[H:2 * H] * cs[s] + s_if[0:H] * t_g
                hs[s] = s_o * jnp.tanh(cs[s])
        rnn_t = jnp.concatenate(hs, axis=1)

        a = alpha_ref[0]
        fused = a * cnn_ref[0] + (1.0 - a) * rnn_t
        z1 = jnp.dot(wfc1_ref[...], fused.astype(bf16),
                     preferred_element_type=f32) + bfc1_ref[...]
        z1 = jnp.maximum(z1, 0.0)
        z2 = jnp.dot(wfc2_ref[...], z1.astype(bf16),
                     preferred_element_type=f32) + bfc2_ref[...]
        out_ref[...] = _sigmoid(z2).reshape(1, 1, TB)

    return body


def kernel(x, w1, b1, w2, b2, w_ih, w_hh, b_ih, b_hh,
           alpha, wfc1, bfc1, wfc2, bfc2):
    B, C_in, L = x.shape
    C1, _, K = w1.shape
    C2 = w2.shape[0]
    H = w_hh.shape[1]
    F1 = wfc1.shape[0]
    pad = K // 2
    Lp = L + 2 * pad

    TB = 2048 if B >= 4096 else max(128, _round_up(B, 128))
    S = max(1, TB // 128)
    B_pad = _round_up(B, TB)
    nt = B_pad // TB

    bf16 = jnp.bfloat16
    f32 = jnp.float32
    xp = jnp.pad(x.astype(bf16), ((0, B_pad - B), (0, 0), (pad, pad)))
    xp = xp.reshape(nt, TB, C_in, Lp)
    x2 = jnp.transpose(xp, (2, 0, 3, 1)).reshape(C_in, nt * Lp * TB)
    x8 = jnp.pad(xp, ((0, 0), (0, 0), (0, 8 - C_in), (0, 0)))
    x8 = jnp.transpose(x8, (2, 0, 3, 1)).reshape(8, nt * Lp * TB)

    w1t = jnp.transpose(w1, (2, 0, 1)).astype(bf16)
    w2t = jnp.transpose(w2, (2, 0, 1)).astype(bf16)
    b1c = b1.reshape(C1, 1)
    b2c = b2.reshape(C2, 1)
    wcat = jnp.concatenate([
        w_ih, jnp.zeros((4 * H, 8 - C_in), f32), w_hh,
        (b_ih + b_hh).reshape(4 * H, 1), jnp.zeros((4 * H, 7), f32),
    ], axis=1).astype(bf16)
    wfc1_b = wfc1.astype(bf16)
    bfc1c = bfc1.reshape(F1, 1)
    wfc2_b = wfc2.astype(bf16)
    bfc2c = bfc2.reshape(1, 1)
    alpha_s = jnp.asarray(alpha, f32).reshape(1)

    conv_body = _make_conv_body(TB, L, pad, K, C1, C2)
    cnn = pl.pallas_call(
        conv_body,
        out_shape=jax.ShapeDtypeStruct((nt, C2, TB), f32),
        grid_spec=pltpu.PrefetchScalarGridSpec(
            num_scalar_prefetch=0,
            grid=(nt,),
            in_specs=[
                pl.BlockSpec((C_in, Lp * TB), lambda i: (0, i)),
                pl.BlockSpec((K, C1, C_in), lambda i: (0, 0, 0)),
                pl.BlockSpec((C1, 1), lambda i: (0, 0)),
                pl.BlockSpec((K, C2, C1), lambda i: (0, 0, 0)),
                pl.BlockSpec((C2, 1), lambda i: (0, 0)),
            ],
            out_specs=pl.BlockSpec((1, C2, TB), lambda i: (i, 0, 0)),
        ),
        compiler_params=pltpu.CompilerParams(
            dimension_semantics=("parallel",),
            vmem_limit_bytes=48 * 1024 * 1024,
        ),
    )(x2, w1t, b1c, w2t, b2c)

    rnn_body = _make_rnn_body(TB, S, L, pad, 8, H, C2, F1)
    out = pl.pallas_call(
        rnn_body,
        out_shape=jax.ShapeDtypeStruct((nt, 1, TB), f32),
        grid_spec=pltpu.PrefetchScalarGridSpec(
            num_scalar_prefetch=0,
            grid=(nt,),
            in_specs=[
                pl.BlockSpec((8, Lp * TB), lambda i: (0, i)),
                pl.BlockSpec((4 * H, 48), lambda i: (0, 0)),
                pl.BlockSpec((1, C2, TB), lambda i: (i, 0, 0)),
                pl.BlockSpec((F1, C2), lambda i: (0, 0)),
                pl.BlockSpec((F1, 1), lambda i: (0, 0)),
                pl.BlockSpec((1, F1), lambda i: (0, 0)),
                pl.BlockSpec((1, 1), lambda i: (0, 0)),
                pl.BlockSpec(memory_space=pltpu.MemorySpace.SMEM),
            ],
            out_specs=pl.BlockSpec((1, 1, TB), lambda i: (i, 0, 0)),
        ),
        compiler_params=pltpu.CompilerParams(
            dimension_semantics=("parallel",),
            vmem_limit_bytes=45 * 1024 * 1024,
        ),
    )(x8, wcat, cnn, wfc1_b, bfc1c, wfc2_b, bfc2c, alpha_s)
    return out.reshape(B_pad, 1)[:B]
